# baseline (speedup 1.0000x reference)
import jax
import jax.numpy as jnp
from jax import lax
from jax.experimental import pallas as pl
from jax.experimental.pallas import tpu as pltpu

_N = 20000
_HW = 256
_W = 16
_B = 8
_C = 2500
_RB = 128
_NCH = 4 * _HW // _RB


def _body(cls_ref, rois_ref, reg_ref,
          scores_ref, mask_ref, boxes_ref,
          m_s, idx_s, base_s, scale_s, acc_s):
    g = pl.program_id(0)

    @pl.when(g < 2)
    def _stage1():
        x = cls_ref[...]
        m = jnp.max(x, axis=0, keepdims=True)
        cell = (lax.broadcasted_iota(jnp.int32, x.shape, 0) + g * _RB)
        idx = jnp.min(jnp.where(x == m, cell, _HW), axis=0, keepdims=True)

        @pl.when(g == 0)
        def _():
            m_s[...] = m
            idx_s[...] = idx

        @pl.when(g == 1)
        def _():
            m0 = m_s[...]
            idx_f = jnp.where(m > m0, idx, idx_s[...])
            m_f = jnp.maximum(m0, m)
            idx_s[...] = idx_f
            scores_ref[...] = m_f
            mask_ref[...] = jnp.where(m_f >= 0.0, 1.0, 0.0)

            fw = (idx_f % _W).astype(jnp.float32)
            fh = (idx_f // _W).astype(jnp.float32)
            r = rois_ref[...]
            x1, y1, x2, y2 = r[0:1], r[1:2], r[2:3], r[3:4]
            cx = (x1 + x2) * 0.5
            cy = (y1 + y2) * 0.5
            hw = (x2 - x1) * 0.5
            hh = (y2 - y1) * 0.5
            nx1 = cx - hw
            ny1 = cy - hh
            bw = ((cx + hw) - nx1) * (1.0 / _W)
            bh = ((cy + hh) - ny1) * (1.0 / _W)
            bx = bw * (fw + 0.5) + nx1
            by = bh * (fh + 0.5) + ny1
            base_s[...] = jnp.concatenate([bx, by, bx, by], axis=0)
            scale_s[...] = jnp.concatenate([bw, bh, bw, bh], axis=0)

    @pl.when(g > 1)
    def _stage2():
        c = g - 2
        k = c // 2
        half = c % 2
        rg = reg_ref[...]
        cell = lax.broadcasted_iota(jnp.int32, rg.shape, 0) + half * _RB
        part = jnp.sum(jnp.where(cell == idx_s[...], rg, 0.0),
                       axis=0, keepdims=True)

        @pl.when(half == 0)
        def _():
            acc_s[...] = part

        @pl.when(half == 1)
        def _():
            off = acc_s[...] + part
            boxes_ref[pl.ds(k, 1)] = (base_s[pl.ds(k, 1)]
                                      + scale_s[pl.ds(k, 1)] * off)


def _fused(cls2, rois4, reg2):
    return pl.pallas_call(
        _body,
        grid=(2 + _NCH,),
        in_specs=[
            pl.BlockSpec((_RB, _N), lambda g: (jnp.minimum(g, 1), 0)),
            pl.BlockSpec((4, _N), lambda g: (0, 0)),
            pl.BlockSpec(
                (_RB, _N),
                lambda g: (jnp.clip(g - 2, 0, _NCH - 1), 0)),
        ],
        out_specs=[
            pl.BlockSpec((1, _N), lambda g: (0, 0)),
            pl.BlockSpec((1, _N), lambda g: (0, 0)),
            pl.BlockSpec((4, _N), lambda g: (0, 0)),
        ],
        out_shape=[
            jax.ShapeDtypeStruct((1, _N), jnp.float32),
            jax.ShapeDtypeStruct((1, _N), jnp.float32),
            jax.ShapeDtypeStruct((4, _N), jnp.float32),
        ],
        scratch_shapes=[
            pltpu.VMEM((1, _N), jnp.float32),
            pltpu.VMEM((1, _N), jnp.int32),
            pltpu.VMEM((4, _N), jnp.float32),
            pltpu.VMEM((4, _N), jnp.float32),
            pltpu.VMEM((1, _N), jnp.float32),
        ],
    )(cls2, rois4, reg2)


def kernel(batch_rois, rcnn_cls_pred, rcnn_reg_pred):
    cls2 = rcnn_cls_pred.reshape(_N, _HW).T
    reg2 = rcnn_reg_pred.reshape(_N, 4 * _HW).T
    rois4 = jnp.transpose(batch_rois, (2, 0, 1)).reshape(4, _N)

    scores2, mask2, boxes4 = _fused(cls2, rois4, reg2)

    boxes = jnp.transpose(boxes4.reshape(4, _B, _C), (1, 2, 0))
    scores = scores2.reshape(_B, _C, 1)
    labels = jnp.zeros_like(scores)
    keep_mask = mask2.astype(jnp.bool_).reshape(_B, _C, 1)
    return boxes, scores, labels, keep_mask


# speedup vs baseline: 1.0691x; 1.0691x over previous
import jax
import jax.numpy as jnp
from jax import lax
from jax.experimental import pallas as pl
from jax.experimental.pallas import tpu as pltpu

_N = 20000
_HW = 256
_W = 16
_B = 8
_C = 2500
_RB = 128
_NCH = 4 * _HW // _RB
_CB = 64
_NCLS = _HW // _CB


def _body(cls_hbm, rois_ref, reg_ref,
          scores_ref, mask_ref, boxes_ref,
          clsbuf, m_s, idx_s, base_s, scale_s, acc_s, sems):
    g = pl.program_id(0)

    @pl.when(g == 0)
    def _stage1():
        def _copy(c):
            return pltpu.make_async_copy(
                cls_hbm.at[pl.ds(c * _CB, _CB), :],
                clsbuf.at[c % 2], sems.at[c % 2])

        _copy(0).start()
        _copy(1).start()
        for c in range(_NCLS):
            _copy(c).wait()
            x = clsbuf[c % 2]
            m = jnp.max(x, axis=0, keepdims=True)
            cell = (lax.broadcasted_iota(jnp.int32, x.shape, 0) + c * _CB)
            idx = jnp.min(jnp.where(x == m, cell, _HW),
                          axis=0, keepdims=True)
            if c == 0:
                m_s[...] = m
                idx_s[...] = idx
            else:
                m0 = m_s[...]
                idx_s[...] = jnp.where(m > m0, idx, idx_s[...])
                m_s[...] = jnp.maximum(m0, m)
            if c + 2 < _NCLS:
                _copy(c + 2).start()

        m_f = m_s[...]
        idx_f = idx_s[...]
        scores_ref[...] = m_f
        mask_ref[...] = jnp.where(m_f >= 0.0, 1.0, 0.0)

        fw = (idx_f % _W).astype(jnp.float32)
        fh = (idx_f // _W).astype(jnp.float32)
        r = rois_ref[...]
        x1, y1, x2, y2 = r[0:1], r[1:2], r[2:3], r[3:4]
        cx = (x1 + x2) * 0.5
        cy = (y1 + y2) * 0.5
        hw = (x2 - x1) * 0.5
        hh = (y2 - y1) * 0.5
        nx1 = cx - hw
        ny1 = cy - hh
        bw = ((cx + hw) - nx1) * (1.0 / _W)
        bh = ((cy + hh) - ny1) * (1.0 / _W)
        bx = bw * (fw + 0.5) + nx1
        by = bh * (fh + 0.5) + ny1
        base_s[...] = jnp.concatenate([bx, by, bx, by], axis=0)
        scale_s[...] = jnp.concatenate([bw, bh, bw, bh], axis=0)

    k = g // 2
    half = g % 2
    rg = reg_ref[...]
    cell = lax.broadcasted_iota(jnp.int32, rg.shape, 0) + half * _RB
    part = jnp.sum(jnp.where(cell == idx_s[...], rg, 0.0),
                   axis=0, keepdims=True)

    @pl.when(half == 0)
    def _():
        acc_s[...] = part

    @pl.when(half == 1)
    def _():
        off = acc_s[...] + part
        boxes_ref[pl.ds(k, 1)] = (base_s[pl.ds(k, 1)]
                                  + scale_s[pl.ds(k, 1)] * off)


def _fused(cls2, rois4, reg2):
    return pl.pallas_call(
        _body,
        grid=(_NCH,),
        in_specs=[
            pl.BlockSpec(memory_space=pl.ANY),
            pl.BlockSpec((4, _N), lambda g: (0, 0)),
            pl.BlockSpec((_RB, _N), lambda g: (g, 0)),
        ],
        out_specs=[
            pl.BlockSpec((1, _N), lambda g: (0, 0)),
            pl.BlockSpec((1, _N), lambda g: (0, 0)),
            pl.BlockSpec((4, _N), lambda g: (0, 0)),
        ],
        out_shape=[
            jax.ShapeDtypeStruct((1, _N), jnp.float32),
            jax.ShapeDtypeStruct((1, _N), jnp.float32),
            jax.ShapeDtypeStruct((4, _N), jnp.float32),
        ],
        scratch_shapes=[
            pltpu.VMEM((2, _CB, _N), jnp.float32),
            pltpu.VMEM((1, _N), jnp.float32),
            pltpu.VMEM((1, _N), jnp.int32),
            pltpu.VMEM((4, _N), jnp.float32),
            pltpu.VMEM((4, _N), jnp.float32),
            pltpu.VMEM((1, _N), jnp.float32),
            pltpu.SemaphoreType.DMA((2,)),
        ],
    )(cls2, rois4, reg2)


def kernel(batch_rois, rcnn_cls_pred, rcnn_reg_pred):
    cls2 = rcnn_cls_pred.reshape(_N, _HW).T
    reg2 = rcnn_reg_pred.reshape(_N, 4 * _HW).T
    rois4 = jnp.transpose(batch_rois, (2, 0, 1)).reshape(4, _N)

    scores2, mask2, boxes4 = _fused(cls2, rois4, reg2)

    boxes = jnp.transpose(boxes4.reshape(4, _B, _C), (1, 2, 0))
    scores = scores2.reshape(_B, _C, 1)
    labels = jnp.zeros_like(scores)
    keep_mask = mask2.astype(jnp.bool_).reshape(_B, _C, 1)
    return boxes, scores, labels, keep_mask
